# baseline (speedup 1.0000x reference)
import functools

import jax
import jax.numpy as jnp
from jax import lax
from jax.experimental import pallas as pl
from jax.experimental.pallas import tpu as pltpu
from jax.experimental.pallas import tpu_sc as plsc

B = 8
P = 1024
N = B * P
K = 16
EPS = 1e-5
NEG_SLOPE = 0.2
DG = 128

NW = 32
PPW = N // NW
CH = 8
NCH = PPW // CH

_DEFAULT = jax.lax.Precision.DEFAULT


def _lrelu(x):
    return jnp.where(x >= 0, x, NEG_SLOPE * x)


def _dot(a, b):
    return jax.lax.dot_general(
        a, b, (((1,), (0,)), ((), ())),
        preferred_element_type=jnp.float32, precision=_DEFAULT)


def _conv_a_body(x_ref, sq_ref, w_ref, b_ref, idx_ref, xit_ref, xpad_ref,
                 *, d, o):
    x = x_ref[...]
    g = jax.lax.dot_general(
        x, x, (((1,), (1,)), ((), ())),
        preferred_element_type=jnp.float32, precision=_DEFAULT)
    sq = sq_ref[...][0, 0, :]
    d2 = sq[:, None] + sq[None, :] - 2.0 * g

    colf = lax.broadcasted_iota(jnp.int32, (P, P), 1).astype(jnp.float32)
    base = pl.program_id(0) * P
    inf = jnp.float32(jnp.inf)
    big = jnp.float32(P)
    cols = []
    for _ in range(K):
        m = jnp.min(d2, axis=1)
        am = jnp.min(jnp.where(d2 == m[:, None], colf, big), axis=1)
        cols.append(am)
        d2 = jnp.where(colf == am[:, None], inf, d2)
    idx_ref[...] = jnp.stack(cols, axis=1).astype(jnp.int32) + base

    xit_ref[...] = _dot(x, w_ref[:d, :]) + b_ref[...]
    if d < DG:
        xpad_ref[...] = jnp.concatenate(
            [x, jnp.zeros((P, DG - d), jnp.float32)], axis=1)
    else:
        xpad_ref[...] = x


def _conv_a(x, w, b):
    d = x.shape[1]
    o = w.shape[1]
    nb = x.shape[0] // P
    xb = x.reshape(nb, P, d)
    sq = jnp.sum(xb * xb, axis=-1).reshape(nb, 1, P)
    return pl.pallas_call(
        functools.partial(_conv_a_body, d=d, o=o),
        grid=(nb,),
        in_specs=[
            pl.BlockSpec((P, d), lambda i: (i, 0)),
            pl.BlockSpec((1, 1, P), lambda i: (i, 0, 0)),
            pl.BlockSpec((2 * d, o), lambda i: (0, 0)),
            pl.BlockSpec((o,), lambda i: (0,)),
        ],
        out_specs=[
            pl.BlockSpec((P, K), lambda i: (i, 0)),
            pl.BlockSpec((P, o), lambda i: (i, 0)),
            pl.BlockSpec((P, DG), lambda i: (i, 0)),
        ],
        out_shape=[
            jax.ShapeDtypeStruct((nb * P, K), jnp.int32),
            jax.ShapeDtypeStruct((nb * P, o), jnp.float32),
            jax.ShapeDtypeStruct((nb * P, DG), jnp.float32),
        ],
    )(x, sq, w, b)


def _gather_sc(xpad, idx3):
    n = xpad.shape[0]
    nch = idx3.shape[1]
    mesh = plsc.VectorSubcoreMesh(core_axis_name="c", subcore_axis_name="s")

    @functools.partial(
        pl.kernel,
        mesh=mesh,
        out_type=jax.ShapeDtypeStruct((n * K, DG), jnp.float32),
        scratch_types=[
            pltpu.VMEM((CH * K,), jnp.int32),
            pltpu.VMEM((CH * K,), jnp.int32),
            pltpu.VMEM((CH * K, DG), jnp.float32),
            pltpu.VMEM((CH * K, DG), jnp.float32),
            pltpu.SemaphoreType.DMA,
            pltpu.SemaphoreType.DMA,
            pltpu.SemaphoreType.DMA,
            pltpu.SemaphoreType.DMA,
        ],
    )
    def k(x_hbm, idx_hbm, out_hbm, idxv0, idxv1, rows0, rows1,
          sg0, sg1, so0, so1):
        wid = lax.axis_index("s") * 2 + lax.axis_index("c")

        def chunk2(u, _):
            t0 = 2 * u
            t1 = t0 + 1
            pltpu.sync_copy(idx_hbm.at[wid, t0], idxv0)
            ga = pltpu.async_copy(x_hbm.at[idxv0], rows0, sg0)
            pltpu.sync_copy(idx_hbm.at[wid, t1], idxv1)
            gb = pltpu.async_copy(x_hbm.at[idxv1], rows1, sg1)
            ga.wait()
            oa = pltpu.async_copy(
                rows0, out_hbm.at[pl.ds((wid * nch + t0) * CH * K, CH * K)],
                so0)
            gb.wait()
            ob = pltpu.async_copy(
                rows1, out_hbm.at[pl.ds((wid * nch + t1) * CH * K, CH * K)],
                so1)
            oa.wait()
            ob.wait()
            return 0

        lax.fori_loop(0, nch // 2, chunk2, 0)

    return k(xpad, idx3)


CBLK = 256


def _conv_c_body(xj_ref, x_ref, xit_ref, wb_ref, out_ref, *, d, o):
    xj = xj_ref[:, :d]
    xi = x_ref[...]
    diff = (xj.reshape(CBLK, K, d) - xi[:, None, :]).reshape(CBLK * K, d)
    e = _dot(diff, wb_ref[...])
    m = jnp.max(e.reshape(CBLK, K, o), axis=1)
    out_ref[...] = xit_ref[...] + m


def _conv_c(xj, x, xit, wb):
    d = x.shape[1]
    o = wb.shape[1]
    n = x.shape[0]
    return pl.pallas_call(
        functools.partial(_conv_c_body, d=d, o=o),
        grid=(n // CBLK,),
        in_specs=[
            pl.BlockSpec((CBLK * K, DG), lambda i: (i, 0)),
            pl.BlockSpec((CBLK, d), lambda i: (i, 0)),
            pl.BlockSpec((CBLK, o), lambda i: (i, 0)),
            pl.BlockSpec((d, o), lambda i: (0, 0)),
        ],
        out_specs=pl.BlockSpec((CBLK, o), lambda i: (i, 0)),
        out_shape=jax.ShapeDtypeStruct((n, o), jnp.float32),
    )(xj, x, xit, wb)


def _edge_conv_pallas(x, w, b):
    d = x.shape[1]
    n = x.shape[0]
    idx, xit, xpad = _conv_a(x, w, b)
    nch = n // (NW * CH)
    xj = _gather_sc(xpad, idx.reshape(NW, nch, CH * K))
    return _conv_c(xj, x, xit, w[d:])


def _head1_body(x1, x2, x3, x4, wl_ref, bl_ref, lin_ref, s_ref, s2_ref):
    lin = jnp.broadcast_to(bl_ref[...][None, :], (P, 1024))
    o0 = 0
    for xr in (x1, x2, x3, x4):
        x = xr[...]
        lin = lin + _dot(x, wl_ref[o0:o0 + x.shape[1], :])
        o0 += x.shape[1]
    lin_ref[...] = lin
    s_ref[...] = jnp.sum(lin, axis=0)[None, None, :]
    s2_ref[...] = jnp.sum(lin * lin, axis=0)[None, None, :]


def _head1(x1, x2, x3, x4, wl, bl):
    splits = (64, 64, 128, 256)
    specs = [pl.BlockSpec((P, oi), lambda i: (i, 0)) for oi in splits]
    specs.append(pl.BlockSpec((512, 1024), lambda i: (0, 0)))
    specs.append(pl.BlockSpec((1024,), lambda i: (0,)))
    return pl.pallas_call(
        _head1_body,
        grid=(B,),
        in_specs=specs,
        out_specs=[
            pl.BlockSpec((P, 1024), lambda i: (i, 0)),
            pl.BlockSpec((1, 1, 1024), lambda i: (i, 0, 0)),
            pl.BlockSpec((1, 1, 1024), lambda i: (i, 0, 0)),
        ],
        out_shape=[
            jax.ShapeDtypeStruct((N, 1024), jnp.float32),
            jax.ShapeDtypeStruct((B, 1, 1024), jnp.float32),
            jax.ShapeDtypeStruct((B, 1, 1024), jnp.float32),
        ],
    )(x1, x2, x3, x4, wl, bl)


def _head2_body(lin_ref, s_ref, s2_ref, gl_ref, betal_ref, pooled_ref):
    s = jnp.sum(s_ref[...], axis=(0, 1))
    s2 = jnp.sum(s2_ref[...], axis=(0, 1))
    mu = s / N
    var = s2 / N - mu * mu
    scale = gl_ref[...] / jnp.sqrt(var + EPS)
    shift = betal_ref[...] - mu * scale
    v = _lrelu(lin_ref[...] * scale[None, :] + shift[None, :])
    pooled_ref[...] = jnp.max(v, axis=0)[None, None, :]


def _head2(lin, s, s2, gl, betal):
    return pl.pallas_call(
        _head2_body,
        grid=(B,),
        in_specs=[
            pl.BlockSpec((P, 1024), lambda i: (i, 0)),
            pl.BlockSpec((B, 1, 1024), lambda i: (0, 0, 0)),
            pl.BlockSpec((B, 1, 1024), lambda i: (0, 0, 0)),
            pl.BlockSpec((1024,), lambda i: (0,)),
            pl.BlockSpec((1024,), lambda i: (0,)),
        ],
        out_specs=pl.BlockSpec((1, 1, 1024), lambda i: (i, 0, 0)),
        out_shape=jax.ShapeDtypeStruct((B, 1, 1024), jnp.float32),
    )(lin, s, s2, gl, betal)


def _bn_rows(x, gamma, beta):
    mu = jnp.mean(x, axis=0, keepdims=True)
    var = jnp.mean((x - mu) * (x - mu), axis=0, keepdims=True)
    return gamma * (x - mu) / jnp.sqrt(var + EPS) + beta


def _head3_body(pooled_ref, wm1_ref, bm1_ref, g1_ref, be1_ref,
                wm2_ref, bm2_ref, g2_ref, be2_ref, wm3_ref, bm3_ref, out_ref):
    h = pooled_ref[...]
    h = _lrelu(_bn_rows(_dot(h, wm1_ref[...]) + bm1_ref[...][None, :],
                        g1_ref[...], be1_ref[...]))
    h = _lrelu(_bn_rows(_dot(h, wm2_ref[...]) + bm2_ref[...][None, :],
                        g2_ref[...], be2_ref[...]))
    out_ref[...] = _dot(h, wm3_ref[...]) + bm3_ref[...][None, :]


def _head3(pooled, wm1, bm1, g1, be1, wm2, bm2, g2, be2, wm3, bm3):
    return pl.pallas_call(
        _head3_body,
        out_shape=jax.ShapeDtypeStruct((B, 40), jnp.float32),
    )(pooled, wm1, bm1, g1, be1, wm2, bm2, g2, be2, wm3, bm3)


NG = 2
GP = N // NG


def kernel(pos, batch, W1, b1, W2, b2, W3, b3, W4, b4, Wl, bl, gl, betal,
           Wm1, bm1, g1, be1, Wm2, bm2, g2, be2, Wm3, bm3):
    xg = [pos[g * GP:(g + 1) * GP] for g in range(NG)]
    feats = []
    for (w, b) in ((W1, b1), (W2, b2), (W3, b3), (W4, b4)):
        xg = [_edge_conv_pallas(x, w, b) for x in xg]
        feats.append(jnp.concatenate(xg, axis=0) if NG > 1 else xg[0])
    x1, x2, x3, x4 = feats
    lin, s, s2 = _head1(x1, x2, x3, x4, Wl, bl)
    pooled = _head2(lin, s, s2, gl, betal).reshape(B, 1024)
    return _head3(pooled, Wm1, bm1, g1, be1, Wm2, bm2, g2, be2, Wm3, bm3)


# speedup vs baseline: 1.1001x; 1.1001x over previous
import functools

import jax
import jax.numpy as jnp
from jax import lax
from jax.experimental import pallas as pl
from jax.experimental.pallas import tpu as pltpu
from jax.experimental.pallas import tpu_sc as plsc

B = 8
P = 1024
N = B * P
K = 16
EPS = 1e-5
NEG_SLOPE = 0.2
DG = 128

NW = 32
PPW = N // NW
CH = 8
NCH = PPW // CH

_DEFAULT = jax.lax.Precision.DEFAULT


def _lrelu(x):
    return jnp.where(x >= 0, x, NEG_SLOPE * x)


def _dot(a, b):
    return jax.lax.dot_general(
        a, b, (((1,), (0,)), ((), ())),
        preferred_element_type=jnp.float32, precision=_DEFAULT)


def _conv_a_body(x_ref, sq_ref, w_ref, b_ref, idx_ref, xit_ref, xpad_ref,
                 *, d, o):
    x = x_ref[...]
    g = jax.lax.dot_general(
        x, x, (((1,), (1,)), ((), ())),
        preferred_element_type=jnp.float32, precision=_DEFAULT)
    sq = sq_ref[...][0, 0, :]
    d2 = sq[:, None] + sq[None, :] - 2.0 * g

    colf = lax.broadcasted_iota(jnp.int32, (P, P), 1).astype(jnp.float32)
    base = pl.program_id(0) * P
    inf = jnp.float32(jnp.inf)
    big = jnp.float32(P)
    cols = []
    for _ in range(K):
        m = jnp.min(d2, axis=1)
        am = jnp.min(jnp.where(d2 == m[:, None], colf, big), axis=1)
        cols.append(am)
        d2 = jnp.where(colf == am[:, None], inf, d2)
    idx_ref[...] = jnp.stack(cols, axis=1).astype(jnp.int32) + base

    xit_ref[...] = _dot(x, w_ref[:d, :]) + b_ref[...]
    if d < DG:
        xpad_ref[...] = jnp.concatenate(
            [x, jnp.zeros((P, DG - d), jnp.float32)], axis=1)
    else:
        xpad_ref[...] = x


def _conv_a(x, w, b):
    d = x.shape[1]
    o = w.shape[1]
    nb = x.shape[0] // P
    xb = x.reshape(nb, P, d)
    sq = jnp.sum(xb * xb, axis=-1).reshape(nb, 1, P)
    return pl.pallas_call(
        functools.partial(_conv_a_body, d=d, o=o),
        grid=(nb,),
        in_specs=[
            pl.BlockSpec((P, d), lambda i: (i, 0)),
            pl.BlockSpec((1, 1, P), lambda i: (i, 0, 0)),
            pl.BlockSpec((2 * d, o), lambda i: (0, 0)),
            pl.BlockSpec((o,), lambda i: (0,)),
        ],
        out_specs=[
            pl.BlockSpec((P, K), lambda i: (i, 0)),
            pl.BlockSpec((P, o), lambda i: (i, 0)),
            pl.BlockSpec((P, DG), lambda i: (i, 0)),
        ],
        out_shape=[
            jax.ShapeDtypeStruct((nb * P, K), jnp.int32),
            jax.ShapeDtypeStruct((nb * P, o), jnp.float32),
            jax.ShapeDtypeStruct((nb * P, DG), jnp.float32),
        ],
    )(x, sq, w, b)


def _gather_sc(xpad, idx3):
    n = xpad.shape[0]
    nch = idx3.shape[1]
    mesh = plsc.VectorSubcoreMesh(core_axis_name="c", subcore_axis_name="s")

    @functools.partial(
        pl.kernel,
        mesh=mesh,
        out_type=jax.ShapeDtypeStruct((n * K, DG), jnp.float32),
        scratch_types=[
            pltpu.VMEM((CH * K,), jnp.int32),
            pltpu.VMEM((CH * K, DG), jnp.float32),
            pltpu.SemaphoreType.DMA,
        ],
    )
    def k(x_hbm, idx_hbm, out_hbm, idxv, rows, sem):
        wid = lax.axis_index("s") * 2 + lax.axis_index("c")

        def chunk(t, _):
            pltpu.sync_copy(idx_hbm.at[wid, t], idxv)
            pltpu.async_copy(x_hbm.at[idxv], rows, sem).wait()
            pltpu.sync_copy(
                rows, out_hbm.at[pl.ds((wid * nch + t) * CH * K, CH * K)])
            return 0

        lax.fori_loop(0, nch, chunk, 0)

    return k(xpad, idx3)


CBLK = 512


def _conv_c_body(xj_ref, x_ref, xit_ref, wb_ref, out_ref, *, d, o):
    xj = xj_ref[:, :d]
    xi = x_ref[...]
    diff = (xj.reshape(CBLK, K, d) - xi[:, None, :]).reshape(CBLK * K, d)
    e = _dot(diff, wb_ref[...])
    m = jnp.max(e.reshape(CBLK, K, o), axis=1)
    out_ref[...] = xit_ref[...] + m


def _conv_c(xj, x, xit, wb):
    d = x.shape[1]
    o = wb.shape[1]
    n = x.shape[0]
    return pl.pallas_call(
        functools.partial(_conv_c_body, d=d, o=o),
        grid=(n // CBLK,),
        in_specs=[
            pl.BlockSpec((CBLK * K, DG), lambda i: (i, 0)),
            pl.BlockSpec((CBLK, d), lambda i: (i, 0)),
            pl.BlockSpec((CBLK, o), lambda i: (i, 0)),
            pl.BlockSpec((d, o), lambda i: (0, 0)),
        ],
        out_specs=pl.BlockSpec((CBLK, o), lambda i: (i, 0)),
        out_shape=jax.ShapeDtypeStruct((n, o), jnp.float32),
    )(xj, x, xit, wb)


def _edge_conv_pallas(x, w, b):
    d = x.shape[1]
    n = x.shape[0]
    idx, xit, xpad = _conv_a(x, w, b)
    nch = n // (NW * CH)
    xj = _gather_sc(xpad, idx.reshape(NW, nch, CH * K))
    return _conv_c(xj, x, xit, w[d:])


def _head1_body(x1, x2, x3, x4, wl_ref, bl_ref, lin_ref, s_ref, s2_ref):
    lin = jnp.broadcast_to(bl_ref[...][None, :], (P, 1024))
    o0 = 0
    for xr in (x1, x2, x3, x4):
        x = xr[...]
        lin = lin + _dot(x, wl_ref[o0:o0 + x.shape[1], :])
        o0 += x.shape[1]
    lin_ref[...] = lin
    s_ref[...] = jnp.sum(lin, axis=0)[None, None, :]
    s2_ref[...] = jnp.sum(lin * lin, axis=0)[None, None, :]


def _head1(x1, x2, x3, x4, wl, bl):
    splits = (64, 64, 128, 256)
    specs = [pl.BlockSpec((P, oi), lambda i: (i, 0)) for oi in splits]
    specs.append(pl.BlockSpec((512, 1024), lambda i: (0, 0)))
    specs.append(pl.BlockSpec((1024,), lambda i: (0,)))
    return pl.pallas_call(
        _head1_body,
        grid=(B,),
        in_specs=specs,
        out_specs=[
            pl.BlockSpec((P, 1024), lambda i: (i, 0)),
            pl.BlockSpec((1, 1, 1024), lambda i: (i, 0, 0)),
            pl.BlockSpec((1, 1, 1024), lambda i: (i, 0, 0)),
        ],
        out_shape=[
            jax.ShapeDtypeStruct((N, 1024), jnp.float32),
            jax.ShapeDtypeStruct((B, 1, 1024), jnp.float32),
            jax.ShapeDtypeStruct((B, 1, 1024), jnp.float32),
        ],
    )(x1, x2, x3, x4, wl, bl)


def _head2_body(lin_ref, s_ref, s2_ref, gl_ref, betal_ref, pooled_ref):
    s = jnp.sum(s_ref[...], axis=(0, 1))
    s2 = jnp.sum(s2_ref[...], axis=(0, 1))
    mu = s / N
    var = s2 / N - mu * mu
    scale = gl_ref[...] / jnp.sqrt(var + EPS)
    shift = betal_ref[...] - mu * scale
    v = _lrelu(lin_ref[...] * scale[None, :] + shift[None, :])
    pooled_ref[...] = jnp.max(v, axis=0)[None, None, :]


def _head2(lin, s, s2, gl, betal):
    return pl.pallas_call(
        _head2_body,
        grid=(B,),
        in_specs=[
            pl.BlockSpec((P, 1024), lambda i: (i, 0)),
            pl.BlockSpec((B, 1, 1024), lambda i: (0, 0, 0)),
            pl.BlockSpec((B, 1, 1024), lambda i: (0, 0, 0)),
            pl.BlockSpec((1024,), lambda i: (0,)),
            pl.BlockSpec((1024,), lambda i: (0,)),
        ],
        out_specs=pl.BlockSpec((1, 1, 1024), lambda i: (i, 0, 0)),
        out_shape=jax.ShapeDtypeStruct((B, 1, 1024), jnp.float32),
    )(lin, s, s2, gl, betal)


def _bn_rows(x, gamma, beta):
    mu = jnp.mean(x, axis=0, keepdims=True)
    var = jnp.mean((x - mu) * (x - mu), axis=0, keepdims=True)
    return gamma * (x - mu) / jnp.sqrt(var + EPS) + beta


def _head3_body(pooled_ref, wm1_ref, bm1_ref, g1_ref, be1_ref,
                wm2_ref, bm2_ref, g2_ref, be2_ref, wm3_ref, bm3_ref, out_ref):
    h = pooled_ref[...]
    h = _lrelu(_bn_rows(_dot(h, wm1_ref[...]) + bm1_ref[...][None, :],
                        g1_ref[...], be1_ref[...]))
    h = _lrelu(_bn_rows(_dot(h, wm2_ref[...]) + bm2_ref[...][None, :],
                        g2_ref[...], be2_ref[...]))
    out_ref[...] = _dot(h, wm3_ref[...]) + bm3_ref[...][None, :]


def _head3(pooled, wm1, bm1, g1, be1, wm2, bm2, g2, be2, wm3, bm3):
    return pl.pallas_call(
        _head3_body,
        out_shape=jax.ShapeDtypeStruct((B, 40), jnp.float32),
    )(pooled, wm1, bm1, g1, be1, wm2, bm2, g2, be2, wm3, bm3)


NG = 2
GP = N // NG


def kernel(pos, batch, W1, b1, W2, b2, W3, b3, W4, b4, Wl, bl, gl, betal,
           Wm1, bm1, g1, be1, Wm2, bm2, g2, be2, Wm3, bm3):
    xg = [pos[g * GP:(g + 1) * GP] for g in range(NG)]
    feats = []
    for (w, b) in ((W1, b1), (W2, b2), (W3, b3), (W4, b4)):
        xg = [_edge_conv_pallas(x, w, b) for x in xg]
        feats.append(jnp.concatenate(xg, axis=0) if NG > 1 else xg[0])
    x1, x2, x3, x4 = feats
    lin, s, s2 = _head1(x1, x2, x3, x4, Wl, bl)
    pooled = _head2(lin, s, s2, gl, betal).reshape(B, 1024)
    return _head3(pooled, Wm1, bm1, g1, be1, Wm2, bm2, g2, be2, Wm3, bm3)
